# Initial kernel scaffold; baseline (speedup 1.0000x reference)
import jax
import jax.numpy as jnp
from jax.experimental import pallas as pl


def kernel(x, gate_w, w_fc, w_proj):
    raise NotImplementedError("write your pallas kernel here")


# speedup vs baseline: 8.7678x; 8.7678x over previous
import jax
import jax.numpy as jnp
from jax.experimental import pallas as pl
from jax.experimental.pallas import tpu as pltpu

E = 64
K = 2
AUX_W = 0.01
T, C, DFF = 2048, 768, 3072
DFF_CHUNK = 512
ND = DFF // DFF_CHUNK


def _moe_dense_body(x_ref, gate_ref, wf_ref, wp_ref, out_ref, aux_ref, w_s):
    e = pl.program_id(0)
    d = pl.program_id(1)

    @pl.when((e == 0) & (d == 0))
    def _router():
        x = x_ref[...]
        logits = jax.lax.dot_general(
            x, gate_ref[...], (((1,), (1,)), ((), ())),
            preferred_element_type=jnp.float32)
        lane = jax.lax.broadcasted_iota(jnp.int32, (T, E), 1)
        m1 = jnp.max(logits, axis=1, keepdims=True)
        i1 = jnp.min(jnp.where(logits == m1, lane, E), axis=1, keepdims=True)
        masked = jnp.where(lane == i1, -jnp.inf, logits)
        m2 = jnp.max(masked, axis=1, keepdims=True)
        i2 = jnp.min(jnp.where(masked == m2, lane, E), axis=1, keepdims=True)
        t = jnp.exp(m2 - m1)
        w1 = 1.0 / (1.0 + t)
        w2 = t * w1
        w_s[...] = (jnp.where(lane == i1, w1, 0.0)
                    + jnp.where(lane == i2, w2, 0.0))
        p = jnp.exp(logits - m1)
        p = p / jnp.sum(p, axis=1, keepdims=True)
        frac = jnp.mean(p, axis=0, keepdims=True)
        aux_ref[...] = (AUX_W * E * jnp.sum(frac * frac)).reshape(1, 1)
        out_ref[...] = jnp.zeros_like(out_ref)

    x = x_ref[...]
    lane = jax.lax.broadcasted_iota(jnp.int32, (T, E), 1)
    wvec = jnp.sum(jnp.where(lane == e, w_s[...], 0.0), axis=1, keepdims=True)
    h = jax.lax.dot_general(
        x, wf_ref[0], (((1,), (0,)), ((), ())),
        preferred_element_type=jnp.float32)
    h = 0.5 * h * (1.0 + jax.lax.erf(h * 0.7071067811865476))
    o = jax.lax.dot_general(
        h, wp_ref[0], (((1,), (0,)), ((), ())),
        preferred_element_type=jnp.float32)
    out_ref[...] += wvec * o


def kernel(x, gate_w, w_fc, w_proj):
    xr = x.reshape(T, C)
    out, aux = pl.pallas_call(
        _moe_dense_body,
        grid=(E, ND),
        in_specs=[
            pl.BlockSpec((T, C), lambda e, d: (0, 0)),
            pl.BlockSpec((E, C), lambda e, d: (0, 0)),
            pl.BlockSpec((1, C, DFF_CHUNK), lambda e, d: (e, 0, d)),
            pl.BlockSpec((1, DFF_CHUNK, C), lambda e, d: (e, d, 0)),
        ],
        out_specs=[
            pl.BlockSpec((T, C), lambda e, d: (0, 0)),
            pl.BlockSpec((1, 1), lambda e, d: (0, 0)),
        ],
        out_shape=[
            jax.ShapeDtypeStruct((T, C), jnp.float32),
            jax.ShapeDtypeStruct((1, 1), jnp.float32),
        ],
        scratch_shapes=[pltpu.VMEM((T, E), jnp.float32)],
        compiler_params=pltpu.CompilerParams(
            dimension_semantics=("arbitrary", "arbitrary")),
    )(xr, gate_w, w_fc, w_proj)
    return out.reshape(x.shape), aux[0, 0]


# speedup vs baseline: 15.4585x; 1.7631x over previous
import functools

import jax
import jax.numpy as jnp
from jax import lax
from jax.experimental import pallas as pl
from jax.experimental.pallas import tpu as pltpu
from jax.experimental.pallas import tpu_sc as plsc

E = 64
AUX_W = 0.01
T, C, DFF = 2048, 768, 3072
DFF_CHUNK = 512
ND = DFF // DFF_CHUNK
BM = 128
NB = T // BM * 2 + E
S = NB * BM
NW = 32
TPW = T // NW
CB = 256


def _router_body(x_ref, gate_ref, posa_ref, posb_ref, wa_ref, wb_ref,
                 sched_ref, aux_ref):
    x = x_ref[...]
    logits = lax.dot_general(x, gate_ref[...], (((1,), (1,)), ((), ())),
                             preferred_element_type=jnp.float32)
    lane = lax.broadcasted_iota(jnp.int32, (T, E), 1)
    m1 = jnp.max(logits, axis=1, keepdims=True)
    i1 = jnp.min(jnp.where(logits == m1, lane, E), axis=1, keepdims=True)
    masked = jnp.where(lane == i1, -jnp.inf, logits)
    m2 = jnp.max(masked, axis=1, keepdims=True)
    i2 = jnp.min(jnp.where(masked == m2, lane, E), axis=1, keepdims=True)
    t = jnp.exp(m2 - m1)
    wa_ref[...] = 1.0 / (1.0 + t)
    wb_ref[...] = t / (1.0 + t)
    p = jnp.exp(logits - m1)
    p = p / jnp.sum(p, axis=1, keepdims=True)
    frac = jnp.mean(p, axis=0, keepdims=True)
    aux_ref[...] = (AUX_W * E * jnp.sum(frac * frac)).reshape(1, 1)

    oh1 = jnp.where(lane == i1, 1.0, 0.0)
    oh2 = jnp.where(lane == i2, 1.0, 0.0)
    ohsum = oh1 + oh2
    rsub = lax.broadcasted_iota(jnp.int32, (CB, CB), 0)
    csub = lax.broadcasted_iota(jnp.int32, (CB, CB), 1)
    tril = jnp.where(csub < rsub, 1.0, 0.0)
    exc_blocks = []
    base = jnp.zeros((1, E), dtype=jnp.float32)
    for rb in range(T // CB):
        mb = ohsum[rb * CB:(rb + 1) * CB, :]
        exc_blocks.append(
            lax.dot_general(tril, mb, (((1,), (0,)), ((), ())),
                            preferred_element_type=jnp.float32) + base)
        base = base + jnp.sum(mb, axis=0, keepdims=True)
    exc = jnp.concatenate(exc_blocks, axis=0)
    counts = base
    nb = jnp.ceil(counts * (1.0 / BM))
    ce1 = lax.broadcasted_iota(jnp.int32, (E, E), 0)
    ce2 = lax.broadcasted_iota(jnp.int32, (E, E), 1)
    upper = jnp.where(ce1 < ce2, 1.0, 0.0)
    bstart = lax.dot_general(nb, upper, (((1,), (0,)), ((), ())),
                             preferred_element_type=jnp.float32)
    sbase = bstart * BM
    posa = jnp.sum(oh1 * (sbase + exc), axis=1, keepdims=True)
    posb = jnp.sum(oh2 * (sbase + exc), axis=1, keepdims=True)
    posa_ref[...] = posa.astype(jnp.int32)
    posb_ref[...] = posb.astype(jnp.int32)
    biota = lax.broadcasted_iota(jnp.int32, (NB, 1), 0).astype(jnp.float32)
    sched = jnp.sum(jnp.where(bstart <= biota, 1.0, 0.0),
                    axis=1, keepdims=True) - 1.0
    total = jnp.sum(nb)
    lane_e = lax.broadcasted_iota(jnp.int32, (1, E), 1).astype(jnp.float32)
    last_used = jnp.max(jnp.where(nb > 0, lane_e, -1.0))
    sched = jnp.where(biota < total, sched, last_used)
    sched_ref[...] = sched.astype(jnp.int32)


def _router(xr, gate_w):
    return pl.pallas_call(
        _router_body,
        grid=(1,),
        in_specs=[
            pl.BlockSpec((T, C), lambda i: (0, 0)),
            pl.BlockSpec((E, C), lambda i: (0, 0)),
        ],
        out_specs=[
            pl.BlockSpec((T, 1), lambda i: (0, 0)),
            pl.BlockSpec((T, 1), lambda i: (0, 0)),
            pl.BlockSpec((T, 1), lambda i: (0, 0)),
            pl.BlockSpec((T, 1), lambda i: (0, 0)),
            pl.BlockSpec((NB, 1), lambda i: (0, 0)),
            pl.BlockSpec((1, 1), lambda i: (0, 0)),
        ],
        out_shape=[
            jax.ShapeDtypeStruct((T, 1), jnp.int32),
            jax.ShapeDtypeStruct((T, 1), jnp.int32),
            jax.ShapeDtypeStruct((T, 1), jnp.float32),
            jax.ShapeDtypeStruct((T, 1), jnp.float32),
            jax.ShapeDtypeStruct((NB, 1), jnp.int32),
            jax.ShapeDtypeStruct((1, 1), jnp.float32),
        ],
    )(xr, gate_w)


_sc_mesh = plsc.VectorSubcoreMesh(core_axis_name="c", subcore_axis_name="s")


@functools.partial(
    pl.kernel,
    mesh=_sc_mesh,
    out_type=jax.ShapeDtypeStruct((S, C), jnp.float32),
    scratch_types=[
        pltpu.VMEM((TPW,), jnp.int32),
        pltpu.VMEM((TPW,), jnp.int32),
        pltpu.VMEM((TPW, C), jnp.float32),
        pltpu.SemaphoreType.DMA,
        pltpu.SemaphoreType.DMA,
    ],
)
def _sc_dispatch(xr_hbm, posa_hbm, posb_hbm, xs_hbm,
                 posa_v, posb_v, rows_v, sema, semb):
    wid = lax.axis_index("s") * 2 + lax.axis_index("c")
    base = wid * TPW
    pltpu.sync_copy(posa_hbm.at[pl.ds(base, TPW)], posa_v)
    pltpu.sync_copy(posb_hbm.at[pl.ds(base, TPW)], posb_v)
    pltpu.sync_copy(xr_hbm.at[pl.ds(base, TPW)], rows_v)
    cpa = pltpu.async_copy(rows_v, xs_hbm.at[posa_v], sema)
    cpb = pltpu.async_copy(rows_v, xs_hbm.at[posb_v], semb)
    cpa.wait()
    cpb.wait()


def _ffn_body(sched_ref, xs_ref, wf_ref, wp_ref, ys_ref):
    d = pl.program_id(1)
    h = lax.dot_general(xs_ref[...], wf_ref[0], (((1,), (0,)), ((), ())),
                        preferred_element_type=jnp.float32)
    h = 0.5 * h * (1.0 + lax.erf(h * 0.7071067811865476))
    o = lax.dot_general(h, wp_ref[0], (((1,), (0,)), ((), ())),
                        preferred_element_type=jnp.float32)

    @pl.when(d == 0)
    def _init():
        ys_ref[...] = o

    @pl.when(d != 0)
    def _acc():
        ys_ref[...] += o


def _ffn(sched, xs, w_fc, w_proj):
    grid_spec = pltpu.PrefetchScalarGridSpec(
        num_scalar_prefetch=1,
        grid=(NB, ND),
        in_specs=[
            pl.BlockSpec((BM, C), lambda b, d, sched: (b, 0)),
            pl.BlockSpec((1, C, DFF_CHUNK), lambda b, d, sched: (sched[b], 0, d)),
            pl.BlockSpec((1, DFF_CHUNK, C), lambda b, d, sched: (sched[b], d, 0)),
        ],
        out_specs=pl.BlockSpec((BM, C), lambda b, d, sched: (b, 0)),
    )
    return pl.pallas_call(
        _ffn_body,
        grid_spec=grid_spec,
        out_shape=jax.ShapeDtypeStruct((S, C), jnp.float32),
        compiler_params=pltpu.CompilerParams(
            dimension_semantics=("arbitrary", "arbitrary")),
    )(sched, xs, w_fc, w_proj)


@functools.partial(
    pl.kernel,
    mesh=_sc_mesh,
    out_type=[
        jax.ShapeDtypeStruct((T, C), jnp.float32),
        jax.ShapeDtypeStruct((T, C), jnp.float32),
    ],
    scratch_types=[
        pltpu.VMEM((TPW,), jnp.int32),
        pltpu.VMEM((TPW,), jnp.int32),
        pltpu.VMEM((TPW, C), jnp.float32),
        pltpu.VMEM((TPW, C), jnp.float32),
        pltpu.SemaphoreType.DMA,
        pltpu.SemaphoreType.DMA,
    ],
)
def _sc_gather_pairs(ys_hbm, posa_hbm, posb_hbm, outa_hbm, outb_hbm,
                     posa_v, posb_v, rowsa_v, rowsb_v, sema, semb):
    wid = lax.axis_index("s") * 2 + lax.axis_index("c")
    base = wid * TPW
    pltpu.sync_copy(posa_hbm.at[pl.ds(base, TPW)], posa_v)
    pltpu.sync_copy(posb_hbm.at[pl.ds(base, TPW)], posb_v)
    cpa = pltpu.async_copy(ys_hbm.at[posa_v], rowsa_v, sema)
    cpb = pltpu.async_copy(ys_hbm.at[posb_v], rowsb_v, semb)
    cpa.wait()
    cpb.wait()
    pltpu.sync_copy(rowsa_v, outa_hbm.at[pl.ds(base, TPW)])
    pltpu.sync_copy(rowsb_v, outb_hbm.at[pl.ds(base, TPW)])


def _combine_body(ra_ref, rb_ref, wa_ref, wb_ref, out_ref):
    out_ref[...] = wa_ref[...] * ra_ref[...] + wb_ref[...] * rb_ref[...]


_CB_T = 512


def _tc_combine(ra, rb, wa, wb):
    return pl.pallas_call(
        _combine_body,
        grid=(T // _CB_T,),
        in_specs=[
            pl.BlockSpec((_CB_T, C), lambda i: (i, 0)),
            pl.BlockSpec((_CB_T, C), lambda i: (i, 0)),
            pl.BlockSpec((_CB_T, 1), lambda i: (i, 0)),
            pl.BlockSpec((_CB_T, 1), lambda i: (i, 0)),
        ],
        out_specs=pl.BlockSpec((_CB_T, C), lambda i: (i, 0)),
        out_shape=jax.ShapeDtypeStruct((T, C), jnp.float32),
    )(ra, rb, wa, wb)


def kernel(x, gate_w, w_fc, w_proj):
    xr = x.reshape(T, C)
    posa, posb, wa, wb, sched, aux = _router(xr, gate_w)
    posa = posa.reshape(T)
    posb = posb.reshape(T)
    xs = _sc_dispatch(xr, posa, posb)
    ys = _ffn(sched.reshape(NB), xs, w_fc, w_proj)
    ra, rb = _sc_gather_pairs(ys, posa, posb)
    out = _tc_combine(ra, rb, wa, wb)
    return out.reshape(x.shape), aux[0, 0]


# speedup vs baseline: 25.6226x; 1.6575x over previous
import functools

import jax
import jax.numpy as jnp
from jax import lax
from jax.experimental import pallas as pl
from jax.experimental.pallas import tpu as pltpu
from jax.experimental.pallas import tpu_sc as plsc

E = 64
AUX_W = 0.01
T, C, DFF = 2048, 768, 3072
DFF_CHUNK = 512
ND = DFF // DFF_CHUNK
BM = 128
NB = T // BM * 2 + E
S = NB * BM
NW = 32
TPW = T // NW
CB = 256


def _router_body(x_ref, gate_ref, posa_ref, posb_ref, wa_ref, wb_ref,
                 sched_ref, aux_ref):
    x = x_ref[...]
    logits = lax.dot_general(x, gate_ref[...], (((1,), (1,)), ((), ())),
                             preferred_element_type=jnp.float32)
    lane = lax.broadcasted_iota(jnp.int32, (T, E), 1)
    m1 = jnp.max(logits, axis=1, keepdims=True)
    i1 = jnp.min(jnp.where(logits == m1, lane, E), axis=1, keepdims=True)
    masked = jnp.where(lane == i1, -jnp.inf, logits)
    m2 = jnp.max(masked, axis=1, keepdims=True)
    i2 = jnp.min(jnp.where(masked == m2, lane, E), axis=1, keepdims=True)
    t = jnp.exp(m2 - m1)
    wa_ref[...] = 1.0 / (1.0 + t)
    wb_ref[...] = t / (1.0 + t)
    p = jnp.exp(logits - m1)
    p = p / jnp.sum(p, axis=1, keepdims=True)
    frac = jnp.mean(p, axis=0, keepdims=True)
    aux_ref[...] = (AUX_W * E * jnp.sum(frac * frac)).reshape(1, 1)

    oh1 = jnp.where(lane == i1, 1.0, 0.0)
    oh2 = jnp.where(lane == i2, 1.0, 0.0)
    ohsum = oh1 + oh2
    rsub = lax.broadcasted_iota(jnp.int32, (CB, CB), 0)
    csub = lax.broadcasted_iota(jnp.int32, (CB, CB), 1)
    tril = jnp.where(csub < rsub, 1.0, 0.0)
    exc_blocks = []
    base = jnp.zeros((1, E), dtype=jnp.float32)
    for rb in range(T // CB):
        mb = ohsum[rb * CB:(rb + 1) * CB, :]
        exc_blocks.append(
            lax.dot_general(tril, mb, (((1,), (0,)), ((), ())),
                            preferred_element_type=jnp.float32) + base)
        base = base + jnp.sum(mb, axis=0, keepdims=True)
    exc = jnp.concatenate(exc_blocks, axis=0)
    counts = base
    nb = jnp.ceil(counts * (1.0 / BM))
    ce1 = lax.broadcasted_iota(jnp.int32, (E, E), 0)
    ce2 = lax.broadcasted_iota(jnp.int32, (E, E), 1)
    upper = jnp.where(ce1 < ce2, 1.0, 0.0)
    bstart = lax.dot_general(nb, upper, (((1,), (0,)), ((), ())),
                             preferred_element_type=jnp.float32)
    sbase = bstart * BM
    posa = jnp.sum(oh1 * (sbase + exc), axis=1, keepdims=True)
    posb = jnp.sum(oh2 * (sbase + exc), axis=1, keepdims=True)
    posa_ref[...] = posa.astype(jnp.int32)
    posb_ref[...] = posb.astype(jnp.int32)
    biota = lax.broadcasted_iota(jnp.int32, (NB, 1), 0).astype(jnp.float32)
    sched = jnp.sum(jnp.where(bstart <= biota, 1.0, 0.0),
                    axis=1, keepdims=True) - 1.0
    total = jnp.sum(nb)
    lane_e = lax.broadcasted_iota(jnp.int32, (1, E), 1).astype(jnp.float32)
    last_used = jnp.max(jnp.where(nb > 0, lane_e, -1.0))
    sched = jnp.where(biota < total, sched, last_used)
    sched_ref[...] = sched.astype(jnp.int32)


def _router(xr, gate_w):
    return pl.pallas_call(
        _router_body,
        grid=(1,),
        in_specs=[
            pl.BlockSpec((T, C), lambda i: (0, 0)),
            pl.BlockSpec((E, C), lambda i: (0, 0)),
        ],
        out_specs=[
            pl.BlockSpec((T, 1), lambda i: (0, 0)),
            pl.BlockSpec((T, 1), lambda i: (0, 0)),
            pl.BlockSpec((T, 1), lambda i: (0, 0)),
            pl.BlockSpec((T, 1), lambda i: (0, 0)),
            pl.BlockSpec((NB, 1), lambda i: (0, 0)),
            pl.BlockSpec((1, 1), lambda i: (0, 0)),
        ],
        out_shape=[
            jax.ShapeDtypeStruct((T, 1), jnp.int32),
            jax.ShapeDtypeStruct((T, 1), jnp.int32),
            jax.ShapeDtypeStruct((T, 1), jnp.float32),
            jax.ShapeDtypeStruct((T, 1), jnp.float32),
            jax.ShapeDtypeStruct((NB, 1), jnp.int32),
            jax.ShapeDtypeStruct((1, 1), jnp.float32),
        ],
    )(xr, gate_w)


_sc_mesh = plsc.VectorSubcoreMesh(core_axis_name="c", subcore_axis_name="s")


@functools.partial(
    pl.kernel,
    mesh=_sc_mesh,
    out_type=jax.ShapeDtypeStruct((S, C), jnp.float32),
    scratch_types=[
        pltpu.VMEM((TPW,), jnp.int32),
        pltpu.VMEM((TPW,), jnp.int32),
        pltpu.VMEM((TPW, C), jnp.float32),
        pltpu.SemaphoreType.DMA,
        pltpu.SemaphoreType.DMA,
    ],
)
def _sc_dispatch(xr_hbm, posa_hbm, posb_hbm, xs_hbm,
                 posa_v, posb_v, rows_v, sema, semb):
    wid = lax.axis_index("s") * 2 + lax.axis_index("c")
    base = wid * TPW
    pltpu.sync_copy(posa_hbm.at[pl.ds(base, TPW)], posa_v)
    pltpu.sync_copy(posb_hbm.at[pl.ds(base, TPW)], posb_v)
    pltpu.sync_copy(xr_hbm.at[pl.ds(base, TPW)], rows_v)
    cpa = pltpu.async_copy(rows_v, xs_hbm.at[posa_v], sema)
    cpb = pltpu.async_copy(rows_v, xs_hbm.at[posb_v], semb)
    cpa.wait()
    cpb.wait()


def _ffn_body(sched_ref, xs_ref, wf_ref, wp_ref, ys_ref):
    acc = jnp.zeros((BM, C), dtype=jnp.float32)
    for d0 in range(0, DFF, DFF_CHUNK):
        h = lax.dot_general(
            xs_ref[...], wf_ref[0, :, d0:d0 + DFF_CHUNK],
            (((1,), (0,)), ((), ())), preferred_element_type=jnp.float32)
        h = 0.5 * h * (1.0 + lax.erf(h * 0.7071067811865476))
        acc = acc + lax.dot_general(
            h, wp_ref[0, d0:d0 + DFF_CHUNK, :],
            (((1,), (0,)), ((), ())), preferred_element_type=jnp.float32)
    ys_ref[...] = acc


def _ffn(sched, xs, w_fc, w_proj):
    grid_spec = pltpu.PrefetchScalarGridSpec(
        num_scalar_prefetch=1,
        grid=(NB,),
        in_specs=[
            pl.BlockSpec((BM, C), lambda b, sched: (b, 0)),
            pl.BlockSpec((1, C, DFF), lambda b, sched: (sched[b], 0, 0)),
            pl.BlockSpec((1, DFF, C), lambda b, sched: (sched[b], 0, 0)),
        ],
        out_specs=pl.BlockSpec((BM, C), lambda b, sched: (b, 0)),
    )
    return pl.pallas_call(
        _ffn_body,
        grid_spec=grid_spec,
        out_shape=jax.ShapeDtypeStruct((S, C), jnp.float32),
        compiler_params=pltpu.CompilerParams(
            dimension_semantics=("arbitrary",)),
    )(sched, xs, w_fc, w_proj)


@functools.partial(
    pl.kernel,
    mesh=_sc_mesh,
    out_type=[
        jax.ShapeDtypeStruct((T, C), jnp.float32),
        jax.ShapeDtypeStruct((T, C), jnp.float32),
    ],
    scratch_types=[
        pltpu.VMEM((TPW,), jnp.int32),
        pltpu.VMEM((TPW,), jnp.int32),
        pltpu.VMEM((TPW, C), jnp.float32),
        pltpu.VMEM((TPW, C), jnp.float32),
        pltpu.SemaphoreType.DMA,
        pltpu.SemaphoreType.DMA,
    ],
)
def _sc_gather_pairs(ys_hbm, posa_hbm, posb_hbm, outa_hbm, outb_hbm,
                     posa_v, posb_v, rowsa_v, rowsb_v, sema, semb):
    wid = lax.axis_index("s") * 2 + lax.axis_index("c")
    base = wid * TPW
    pltpu.sync_copy(posa_hbm.at[pl.ds(base, TPW)], posa_v)
    pltpu.sync_copy(posb_hbm.at[pl.ds(base, TPW)], posb_v)
    cpa = pltpu.async_copy(ys_hbm.at[posa_v], rowsa_v, sema)
    cpb = pltpu.async_copy(ys_hbm.at[posb_v], rowsb_v, semb)
    cpa.wait()
    cpb.wait()
    pltpu.sync_copy(rowsa_v, outa_hbm.at[pl.ds(base, TPW)])
    pltpu.sync_copy(rowsb_v, outb_hbm.at[pl.ds(base, TPW)])


def _combine_body(ra_ref, rb_ref, wa_ref, wb_ref, out_ref):
    out_ref[...] = wa_ref[...] * ra_ref[...] + wb_ref[...] * rb_ref[...]


_CB_T = 512


def _tc_combine(ra, rb, wa, wb):
    return pl.pallas_call(
        _combine_body,
        grid=(T // _CB_T,),
        in_specs=[
            pl.BlockSpec((_CB_T, C), lambda i: (i, 0)),
            pl.BlockSpec((_CB_T, C), lambda i: (i, 0)),
            pl.BlockSpec((_CB_T, 1), lambda i: (i, 0)),
            pl.BlockSpec((_CB_T, 1), lambda i: (i, 0)),
        ],
        out_specs=pl.BlockSpec((_CB_T, C), lambda i: (i, 0)),
        out_shape=jax.ShapeDtypeStruct((T, C), jnp.float32),
    )(ra, rb, wa, wb)


def kernel(x, gate_w, w_fc, w_proj):
    xr = x.reshape(T, C)
    posa, posb, wa, wb, sched, aux = _router(xr, gate_w)
    posa = posa.reshape(T)
    posb = posb.reshape(T)
    xs = _sc_dispatch(xr, posa, posb)
    ys = _ffn(sched.reshape(NB), xs, w_fc, w_proj)
    ra, rb = _sc_gather_pairs(ys, posa, posb)
    out = _tc_combine(ra, rb, wa, wb)
    return out.reshape(x.shape), aux[0, 0]


# speedup vs baseline: 26.0376x; 1.0162x over previous
import functools

import jax
import jax.numpy as jnp
from jax import lax
from jax.experimental import pallas as pl
from jax.experimental.pallas import tpu as pltpu
from jax.experimental.pallas import tpu_sc as plsc

E = 64
AUX_W = 0.01
T, C, DFF = 2048, 768, 3072
DFF_CHUNK = 512
ND = DFF // DFF_CHUNK
BM = 128
NB = T // BM * 2 + E
S = NB * BM
NW = 32
TPW = T // NW
CB = 256


def _router_body(x_ref, gate_ref, posa_ref, posb_ref, wa_ref, wb_ref,
                 sched_ref, aux_ref):
    x = x_ref[...]
    logits = lax.dot_general(x, gate_ref[...], (((1,), (1,)), ((), ())),
                             preferred_element_type=jnp.float32)
    lane = lax.broadcasted_iota(jnp.int32, (T, E), 1)
    m1 = jnp.max(logits, axis=1, keepdims=True)
    i1 = jnp.min(jnp.where(logits == m1, lane, E), axis=1, keepdims=True)
    masked = jnp.where(lane == i1, -jnp.inf, logits)
    m2 = jnp.max(masked, axis=1, keepdims=True)
    i2 = jnp.min(jnp.where(masked == m2, lane, E), axis=1, keepdims=True)
    t = jnp.exp(m2 - m1)
    wa_ref[...] = 1.0 / (1.0 + t)
    wb_ref[...] = t / (1.0 + t)
    p = jnp.exp(logits - m1)
    p = p / jnp.sum(p, axis=1, keepdims=True)
    frac = jnp.mean(p, axis=0, keepdims=True)
    aux_ref[...] = (AUX_W * E * jnp.sum(frac * frac)).reshape(1, 1)

    oh1 = jnp.where(lane == i1, 1.0, 0.0)
    oh2 = jnp.where(lane == i2, 1.0, 0.0)
    ohsum = oh1 + oh2
    rsub = lax.broadcasted_iota(jnp.int32, (CB, CB), 0)
    csub = lax.broadcasted_iota(jnp.int32, (CB, CB), 1)
    tril = jnp.where(csub < rsub, 1.0, 0.0)
    exc_blocks = []
    base = jnp.zeros((1, E), dtype=jnp.float32)
    for rb in range(T // CB):
        mb = ohsum[rb * CB:(rb + 1) * CB, :]
        exc_blocks.append(
            lax.dot_general(tril, mb, (((1,), (0,)), ((), ())),
                            preferred_element_type=jnp.float32) + base)
        base = base + jnp.sum(mb, axis=0, keepdims=True)
    exc = jnp.concatenate(exc_blocks, axis=0)
    counts = base
    nb = jnp.ceil(counts * (1.0 / BM))
    ce1 = lax.broadcasted_iota(jnp.int32, (E, E), 0)
    ce2 = lax.broadcasted_iota(jnp.int32, (E, E), 1)
    upper = jnp.where(ce1 < ce2, 1.0, 0.0)
    bstart = lax.dot_general(nb, upper, (((1,), (0,)), ((), ())),
                             preferred_element_type=jnp.float32)
    sbase = bstart * BM
    posa = jnp.sum(oh1 * (sbase + exc), axis=1, keepdims=True)
    posb = jnp.sum(oh2 * (sbase + exc), axis=1, keepdims=True)
    posa_ref[...] = posa.astype(jnp.int32)
    posb_ref[...] = posb.astype(jnp.int32)
    biota = lax.broadcasted_iota(jnp.int32, (NB, 1), 0).astype(jnp.float32)
    sched = jnp.sum(jnp.where(bstart <= biota, 1.0, 0.0),
                    axis=1, keepdims=True) - 1.0
    total = jnp.sum(nb)
    lane_e = lax.broadcasted_iota(jnp.int32, (1, E), 1).astype(jnp.float32)
    last_used = jnp.max(jnp.where(nb > 0, lane_e, -1.0))
    sched = jnp.where(biota < total, sched, last_used)
    sched_ref[...] = sched.astype(jnp.int32)


def _router(xr, gate_w):
    return pl.pallas_call(
        _router_body,
        grid=(1,),
        in_specs=[
            pl.BlockSpec((T, C), lambda i: (0, 0)),
            pl.BlockSpec((E, C), lambda i: (0, 0)),
        ],
        out_specs=[
            pl.BlockSpec((T, 1), lambda i: (0, 0)),
            pl.BlockSpec((T, 1), lambda i: (0, 0)),
            pl.BlockSpec((T, 1), lambda i: (0, 0)),
            pl.BlockSpec((T, 1), lambda i: (0, 0)),
            pl.BlockSpec((NB, 1), lambda i: (0, 0)),
            pl.BlockSpec((1, 1), lambda i: (0, 0)),
        ],
        out_shape=[
            jax.ShapeDtypeStruct((T, 1), jnp.int32),
            jax.ShapeDtypeStruct((T, 1), jnp.int32),
            jax.ShapeDtypeStruct((T, 1), jnp.float32),
            jax.ShapeDtypeStruct((T, 1), jnp.float32),
            jax.ShapeDtypeStruct((NB, 1), jnp.int32),
            jax.ShapeDtypeStruct((1, 1), jnp.float32),
        ],
    )(xr, gate_w)


_sc_mesh = plsc.VectorSubcoreMesh(core_axis_name="c", subcore_axis_name="s")


@functools.partial(
    pl.kernel,
    mesh=_sc_mesh,
    out_type=jax.ShapeDtypeStruct((S, C), jnp.float32),
    scratch_types=[
        pltpu.VMEM((TPW,), jnp.int32),
        pltpu.VMEM((TPW,), jnp.int32),
        pltpu.VMEM((TPW, C), jnp.float32),
        pltpu.SemaphoreType.DMA,
        pltpu.SemaphoreType.DMA,
    ],
)
def _sc_dispatch(xr_hbm, posa_hbm, posb_hbm, xs_hbm,
                 posa_v, posb_v, rows_v, sema, semb):
    wid = lax.axis_index("s") * 2 + lax.axis_index("c")
    base = wid * TPW
    pltpu.sync_copy(posa_hbm.at[pl.ds(base, TPW)], posa_v)
    pltpu.sync_copy(posb_hbm.at[pl.ds(base, TPW)], posb_v)
    pltpu.sync_copy(xr_hbm.at[pl.ds(base, TPW)], rows_v)
    cpa = pltpu.async_copy(rows_v, xs_hbm.at[posa_v], sema)
    cpb = pltpu.async_copy(rows_v, xs_hbm.at[posb_v], semb)
    cpa.wait()
    cpb.wait()


_CH = C // 2
_DH = DFF // 2


def _ffn_body(sched_ref, xs_ref, wfa_ref, wfb_ref, wpa_ref, wpb_ref, ys_ref):
    x = xs_ref[...]
    h = (lax.dot_general(x[:, :_CH], wfa_ref[0], (((1,), (0,)), ((), ())),
                         preferred_element_type=jnp.float32)
         + lax.dot_general(x[:, _CH:], wfb_ref[0], (((1,), (0,)), ((), ())),
                           preferred_element_type=jnp.float32))
    h = 0.5 * h * (1.0 + lax.erf(h * 0.7071067811865476))
    ys_ref[...] = (
        lax.dot_general(h[:, :_DH], wpa_ref[0], (((1,), (0,)), ((), ())),
                        preferred_element_type=jnp.float32)
        + lax.dot_general(h[:, _DH:], wpb_ref[0], (((1,), (0,)), ((), ())),
                          preferred_element_type=jnp.float32))


def _ffn(sched, xs, w_fc, w_proj):
    grid_spec = pltpu.PrefetchScalarGridSpec(
        num_scalar_prefetch=1,
        grid=(NB,),
        in_specs=[
            pl.BlockSpec((BM, C), lambda b, sched: (b, 0)),
            pl.BlockSpec((1, _CH, DFF), lambda b, sched: (sched[b], 0, 0)),
            pl.BlockSpec((1, _CH, DFF), lambda b, sched: (sched[b], 1, 0)),
            pl.BlockSpec((1, _DH, C), lambda b, sched: (sched[b], 0, 0)),
            pl.BlockSpec((1, _DH, C), lambda b, sched: (sched[b], 1, 0)),
        ],
        out_specs=pl.BlockSpec((BM, C), lambda b, sched: (b, 0)),
    )
    return pl.pallas_call(
        _ffn_body,
        grid_spec=grid_spec,
        out_shape=jax.ShapeDtypeStruct((S, C), jnp.float32),
        compiler_params=pltpu.CompilerParams(
            dimension_semantics=("arbitrary",)),
    )(sched, xs, w_fc, w_fc, w_proj, w_proj)


@functools.partial(
    pl.kernel,
    mesh=_sc_mesh,
    out_type=[
        jax.ShapeDtypeStruct((T, C), jnp.float32),
        jax.ShapeDtypeStruct((T, C), jnp.float32),
    ],
    scratch_types=[
        pltpu.VMEM((TPW,), jnp.int32),
        pltpu.VMEM((TPW,), jnp.int32),
        pltpu.VMEM((TPW, C), jnp.float32),
        pltpu.VMEM((TPW, C), jnp.float32),
        pltpu.SemaphoreType.DMA,
        pltpu.SemaphoreType.DMA,
    ],
)
def _sc_gather_pairs(ys_hbm, posa_hbm, posb_hbm, outa_hbm, outb_hbm,
                     posa_v, posb_v, rowsa_v, rowsb_v, sema, semb):
    wid = lax.axis_index("s") * 2 + lax.axis_index("c")
    base = wid * TPW
    pltpu.sync_copy(posa_hbm.at[pl.ds(base, TPW)], posa_v)
    pltpu.sync_copy(posb_hbm.at[pl.ds(base, TPW)], posb_v)
    cpa = pltpu.async_copy(ys_hbm.at[posa_v], rowsa_v, sema)
    cpb = pltpu.async_copy(ys_hbm.at[posb_v], rowsb_v, semb)
    cpa.wait()
    cpb.wait()
    pltpu.sync_copy(rowsa_v, outa_hbm.at[pl.ds(base, TPW)])
    pltpu.sync_copy(rowsb_v, outb_hbm.at[pl.ds(base, TPW)])


def _combine_body(ra_ref, rb_ref, wa_ref, wb_ref, out_ref):
    out_ref[...] = wa_ref[...] * ra_ref[...] + wb_ref[...] * rb_ref[...]


_CB_T = 512


def _tc_combine(ra, rb, wa, wb):
    return pl.pallas_call(
        _combine_body,
        grid=(T // _CB_T,),
        in_specs=[
            pl.BlockSpec((_CB_T, C), lambda i: (i, 0)),
            pl.BlockSpec((_CB_T, C), lambda i: (i, 0)),
            pl.BlockSpec((_CB_T, 1), lambda i: (i, 0)),
            pl.BlockSpec((_CB_T, 1), lambda i: (i, 0)),
        ],
        out_specs=pl.BlockSpec((_CB_T, C), lambda i: (i, 0)),
        out_shape=jax.ShapeDtypeStruct((T, C), jnp.float32),
    )(ra, rb, wa, wb)


def kernel(x, gate_w, w_fc, w_proj):
    xr = x.reshape(T, C)
    posa, posb, wa, wb, sched, aux = _router(xr, gate_w)
    posa = posa.reshape(T)
    posb = posb.reshape(T)
    xs = _sc_dispatch(xr, posa, posb)
    ys = _ffn(sched.reshape(NB), xs, w_fc, w_proj)
    ra, rb = _sc_gather_pairs(ys, posa, posb)
    out = _tc_combine(ra, rb, wa, wb)
    return out.reshape(x.shape), aux[0, 0]
